# baseline (speedup 1.0000x reference)
import jax
import jax.numpy as jnp
from jax.experimental import pallas as pl
from jax.experimental.pallas import tpu as pltpu

_BLK = 8000


def _body(gu_ref, fi_ref, out_ref):
    prod = gu_ref[...] * fi_ref[...]
    out_ref[...] = jnp.sum(prod, axis=1).reshape(1, 1, -1)


def kernel(gu, fi):
    B, D = gu.shape
    grid = B // _BLK
    out = pl.pallas_call(
        _body,
        grid=(grid,),
        in_specs=[
            pl.BlockSpec((_BLK, D), lambda i: (i, 0)),
            pl.BlockSpec((_BLK, D), lambda i: (i, 0)),
        ],
        out_specs=pl.BlockSpec((1, 1, _BLK), lambda i: (i, 0, 0)),
        out_shape=jax.ShapeDtypeStruct((grid, 1, _BLK), jnp.float32),
    )(gu, fi)
    return out.reshape(B)
